# baseline (speedup 1.0000x reference)
import jax
import jax.numpy as jnp
from jax import lax
from jax.experimental import pallas as pl
from jax.experimental.pallas import tpu as pltpu
from jax.experimental.pallas import tpu_sc as plsc

_NV = 999
_NVA = 1024
_T = 50
_TA = 50
_TP = 64
_BATCH = 1024
_NW = 32
_BPW = _BATCH // _NW


def _sc_body(in_hbm, out_hbm, buf, vals):
    wid = lax.axis_index("s") * 2 + lax.axis_index("c")

    pltpu.sync_copy(in_hbm.at[pl.ds(wid * _BPW * _TP, _BPW * _TP)], vals)

    zeros16 = jnp.zeros((16,), jnp.float32)
    ones16 = jnp.ones((16,), jnp.float32)
    iota16 = lax.iota(jnp.int32, 16)

    def _zero_row(r):
        for j in range(_NVA // 16):
            buf[r, pl.ds(j * 16, 16)] = zeros16

    pl.loop(0, _TA)(_zero_row)

    def _scatter(c, value_vec):
        for j in range(4):
            rows = iota16 + (16 * j)
            v = vals[pl.ds(c * _TP + 16 * j, 16)]
            col = jnp.maximum(v - 1, 0)
            m = (rows < _T) & (v > 0)
            plsc.store_scatter(buf, [rows, col], value_vec, mask=m)

    def _chunk(c):
        b = wid * _BPW + c
        _scatter(c, ones16)
        pltpu.sync_copy(buf, out_hbm.at[b])
        _scatter(c, zeros16)

    pl.loop(0, _BPW)(_chunk)


def kernel(inputs):
    padded = jnp.zeros((_BATCH, _TP), jnp.int32).at[:, :_T].set(inputs)
    flat = padded.reshape(_BATCH * _TP)
    mesh = plsc.VectorSubcoreMesh(core_axis_name="c", subcore_axis_name="s")
    out = pl.kernel(
        _sc_body,
        out_type=jax.ShapeDtypeStruct((_BATCH, _TA, _NVA), jnp.float32),
        mesh=mesh,
        compiler_params=pltpu.CompilerParams(
            use_tc_tiling_on_sc=False, needs_layout_passes=False
        ),
        scratch_types=[
            pltpu.VMEM((_TA, _NVA), jnp.float32),
            pltpu.VMEM((_BPW * _TP,), jnp.int32),
        ],
    )(flat)
    return out[:, :_T, :_NV]


# speedup vs baseline: 1.8287x; 1.8287x over previous
import jax
import jax.numpy as jnp
from jax import lax
from jax.experimental import pallas as pl
from jax.experimental.pallas import tpu as pltpu
from jax.experimental.pallas import tpu_sc as plsc

_NV = 999
_NVA = 1024
_T = 50
_TA = 56
_TP = 64
_BATCH = 1024
_NW = 32
_BPW = _BATCH // _NW


def _sc_body(in_hbm, out_hbm, buf, vals):
    wid = lax.axis_index("s") * 2 + lax.axis_index("c")

    pltpu.sync_copy(in_hbm.at[pl.ds(wid * _BPW * _TP, _BPW * _TP)], vals)

    zeros16 = jnp.zeros((16,), jnp.float32)
    ones16 = jnp.ones((16,), jnp.float32)
    iota16 = lax.iota(jnp.int32, 16)

    def _zero_row(r):
        for j in range(_NVA // 16):
            buf[r, pl.ds(j * 16, 16)] = zeros16

    pl.loop(0, _TA)(_zero_row)

    def _scatter(c, value_vec):
        for j in range(4):
            rows = iota16 + (16 * j)
            v = vals[pl.ds(c * _TP + 16 * j, 16)]
            col = jnp.maximum(v - 1, 0)
            m = (rows < _T) & (v > 0)
            plsc.store_scatter(buf, [rows, col], value_vec, mask=m)

    def _chunk(c):
        b = wid * _BPW + c
        _scatter(c, ones16)
        pltpu.sync_copy(buf, out_hbm.at[b])
        _scatter(c, zeros16)

    pl.loop(0, _BPW)(_chunk)


def kernel(inputs):
    padded = jnp.zeros((_BATCH, _TP), jnp.int32).at[:, :_T].set(inputs)
    flat = padded.reshape(_BATCH * _TP)
    mesh = plsc.VectorSubcoreMesh(core_axis_name="c", subcore_axis_name="s")
    out = pl.kernel(
        _sc_body,
        out_type=jax.ShapeDtypeStruct((_BATCH, _TA, _NVA), jnp.float32),
        mesh=mesh,
        compiler_params=pltpu.CompilerParams(
            use_tc_tiling_on_sc=True, needs_layout_passes=False
        ),
        scratch_types=[
            pltpu.VMEM((_TA, _NVA), jnp.float32),
            pltpu.VMEM((_BPW * _TP,), jnp.int32),
        ],
    )(flat)
    return out[:, :_T, :_NV]
